# Initial kernel scaffold; baseline (speedup 1.0000x reference)
import jax
import jax.numpy as jnp
from jax.experimental import pallas as pl


def kernel(nodes, edges, batch, W1, b1, W2, b2, W3, b3, W4, b4, W5, b5, att, w_fit, b_fit):
    raise NotImplementedError("write your pallas kernel here")


# speedup vs baseline: 1.0059x; 1.0059x over previous
import functools

import jax
import jax.numpy as jnp
from jax.experimental import pallas as pl
from jax.experimental.pallas import tpu as pltpu

_N = 10000
_E = 160000
_FIN = 128
_H = 64
_K = 500
_KP = 512

_BM = 2000
_BR = 400


def _mm_r_kernel(s_ref, ap_ref, o_ref):
    o_ref[...] = jnp.dot(s_ref[...], ap_ref[...],
                         preferred_element_type=jnp.float32)


def _mm_big_kernel(r_ref, st_ref, o_ref):
    o_ref[...] = jnp.dot(r_ref[...], st_ref[...],
                         preferred_element_type=jnp.float32)


def _reconstruct_adj(s_pad, st_pad, adj_pool_pad):
    r = pl.pallas_call(
        _mm_r_kernel,
        grid=(_N // _BM,),
        in_specs=[
            pl.BlockSpec((_BM, _KP), lambda i: (i, 0)),
            pl.BlockSpec((_KP, _KP), lambda i: (0, 0)),
        ],
        out_specs=pl.BlockSpec((_BM, _KP), lambda i: (i, 0)),
        out_shape=jax.ShapeDtypeStruct((_N, _KP), jnp.float32),
    )(s_pad, adj_pool_pad)
    r16 = r.astype(jnp.bfloat16)
    adj = pl.pallas_call(
        _mm_big_kernel,
        grid=(_N // _BR,),
        in_specs=[
            pl.BlockSpec((_BR, _KP), lambda i: (i, 0)),
            pl.BlockSpec((_KP, _N), lambda i: (0, 0)),
        ],
        out_specs=pl.BlockSpec((_BR, _N), lambda i: (i, 0)),
        out_shape=jax.ShapeDtypeStruct((_N, _N), jnp.float32),
    )(r16, st_pad)
    return adj


def _gcn(x, s2, d2, dinv, W, b):
    coef = (dinv[s2] * dinv[d2])[:, None]
    msg = (x @ W)[s2] * coef
    return jax.ops.segment_sum(msg, d2, num_segments=_N) + b


def kernel(nodes, edges, batch, W1, b1, W2, b2, W3, b3, W4, b4, W5, b5,
           att, w_fit, b_fit):
    src, dst = edges[0], edges[1]
    loop = jnp.arange(_N)
    s2 = jnp.concatenate([src, loop])
    d2 = jnp.concatenate([dst, loop])
    deg = jnp.zeros((_N,), jnp.float32).at[d2].add(1.0)
    dinv = jnp.where(deg > 0, 1.0 / jnp.sqrt(deg), 0.0)

    x = jnp.tanh(_gcn(nodes, s2, d2, dinv, W1, b1))
    x = jnp.tanh(_gcn(x, s2, d2, dinv, W2, b2))

    x_q = jax.ops.segment_max(x[s2], d2, num_segments=_N)
    raw = jax.nn.leaky_relu(
        jnp.concatenate([x_q[d2], x[s2]], axis=1) @ att, 0.2)
    m = jax.ops.segment_max(raw, d2, num_segments=_N)
    ex = jnp.exp(raw - m[d2])
    den = jax.ops.segment_sum(ex, d2, num_segments=_N)
    score = ex / (den[d2] + 1e-16)
    x_c = jax.ops.segment_sum(score[:, None] * x[s2], d2, num_segments=_N)
    fitness = jax.nn.sigmoid(x_c @ w_fit + b_fit)
    vals, perm = jax.lax.top_k(fitness, _K)
    x_pool = x_c[perm] * vals[:, None]
    inv = jnp.full((_N,), _K, jnp.int32).at[perm].set(
        jnp.arange(_K, dtype=jnp.int32))
    col = inv[d2]
    s_sel = jnp.zeros((_N, _K + 1), jnp.float32).at[s2, col].add(score)[:, :_K]
    a_s = jax.ops.segment_sum(s_sel[dst], src, num_segments=_N)
    adj_pool = s_sel.T @ a_s

    x_out = s_sel @ x_pool

    s_pad = jnp.pad(s_sel, ((0, 0), (0, _KP - _K)))
    st_pad = jnp.pad(s_sel.T, ((0, _KP - _K), (0, 0))).astype(jnp.bfloat16)
    ap_pad = jnp.pad(adj_pool, ((0, _KP - _K), (0, _KP - _K)))
    adj_out = _reconstruct_adj(s_pad, st_pad, ap_pad)

    x = jnp.tanh(_gcn(x_out, s2, d2, dinv, W3, b3))
    x = jnp.tanh(_gcn(x, s2, d2, dinv, W4, b4))
    x = _gcn(x, s2, d2, dinv, W5, b5)
    return x, adj_out[None]


# speedup vs baseline: 1.4956x; 1.4869x over previous
import functools

import jax
import jax.numpy as jnp
from jax import lax
from jax.experimental import pallas as pl
from jax.experimental.pallas import tpu as pltpu
from jax.experimental.pallas import tpu_sc as plsc

_N = 10000
_E = 160000
_FIN = 128
_H = 64
_K = 500
_KP = 512
_NP = 10240
_TRASH = 10100

_CH = 512
_NW = 32
_EP1 = _NW * _CH * 11

_BM = 2000
_BR = 400


def _mesh():
    return plsc.VectorSubcoreMesh(core_axis_name="c", subcore_axis_name="s")


_NH = _NP // 2
_AH = 5248


def _make_row_pass(ep, steps, weighted):
    rpt = _AH // 16
    zrows = 82
    d = 128

    scratch = [
        pltpu.VMEM((_CH,), jnp.int32),
        pltpu.VMEM((_CH,), jnp.int32),
        pltpu.VMEM((_CH,), jnp.int32),
        pltpu.VMEM((_CH, d), jnp.float32),
        pltpu.VMEM((zrows, d), jnp.float32),
        pltpu.VMEM_SHARED((_AH, d), jnp.float32),
        pltpu.SemaphoreType.DMA,
    ]
    if weighted:
        scratch.append(pltpu.VMEM((_CH,), jnp.float32))

    def body(h_hbm, s_hbm, dst_hbm, w_hbm, out_hbm, idx_s, idx_d, idx_l,
             rows, zb, acc, sem, wbuf=None):
        c = lax.axis_index("c")
        s = lax.axis_index("s")
        wid = s * 2 + c
        zero = jnp.zeros((16,), jnp.float32)

        def zloop(i, carry):
            for k in range(d // 16):
                zb[i, pl.ds(k * 16, 16)] = zero
            return carry

        for phase in range(2):
            lo = phase * _NH
            lax.fori_loop(0, zrows, zloop, 0)
            for blk in range(rpt // zrows):
                pltpu.sync_copy(zb,
                                acc.at[pl.ds(s * rpt + blk * zrows, zrows)])
            plsc.subcore_barrier()

            def step(i, carry):
                off = (wid * steps + i) * _CH
                pltpu.sync_copy(s_hbm.at[pl.ds(off, _CH)], idx_s)
                pltpu.sync_copy(dst_hbm.at[pl.ds(off, _CH)], idx_d)

                def remap(g, carry2):
                    sl = pl.ds(g * 16, 16)
                    d16 = idx_d[sl]
                    inph = (d16 >= lo) & (d16 < lo + _NH)
                    idx_l[sl] = jnp.where(inph, d16 - lo, jnp.int32(_NH))
                    return carry2

                lax.fori_loop(0, _CH // 16, remap, 0)
                pltpu.async_copy(h_hbm.at[idx_s], rows, sem).wait()
                if weighted:
                    pltpu.sync_copy(w_hbm.at[pl.ds(off, _CH)], wbuf)

                    def scale(g, carry2):
                        w16 = wbuf[pl.ds(g * 16, 16)]
                        for j in range(16):
                            wr = w16[j]
                            for k in range(d // 16):
                                sl = pl.ds(k * 16, 16)
                                rows[g * 16 + j, sl] = rows[g * 16 + j,
                                                            sl] * wr
                        return carry2

                    lax.fori_loop(0, _CH // 16, scale, 0)
                pltpu.sync_copy(rows, acc.at[idx_l], add=True)
                return carry

            lax.fori_loop(0, steps, step, 0)
            plsc.subcore_barrier()
            pltpu.sync_copy(acc.at[pl.ds(s * rpt, rpt)],
                            out_hbm.at[c, phase, pl.ds(s * rpt, rpt)])

    out_t = jax.ShapeDtypeStruct((2, 2, _AH, d), jnp.float32)
    if weighted:
        def body_w(h, srcr, dstr, w, out, idx_s, idx_d, idx_l, rows, zb,
                   acc, sem, wbuf):
            body(h, srcr, dstr, w, out, idx_s, idx_d, idx_l, rows, zb, acc,
                 sem, wbuf)

        return pl.kernel(body_w, out_type=out_t, mesh=_mesh(),
                         scratch_types=scratch)

    def body_u(h, srcr, dstr, out, idx_s, idx_d, idx_l, rows, zb, acc, sem):
        body(h, srcr, dstr, None, out, idx_s, idx_d, idx_l, rows, zb, acc,
             sem)

    return pl.kernel(body_u, out_type=out_t, mesh=_mesh(),
                     scratch_types=scratch)


def _row_partials(out4):
    return jnp.concatenate([out4[:, 0, :_NH], out4[:, 1, :_NH]], axis=1)


def _make_scalar_pass(ep, steps):
    scratch = [
        pltpu.VMEM((_CH,), jnp.int32),
        pltpu.VMEM((_CH,), jnp.float32),
        pltpu.VMEM((_NP // 16,), jnp.float32),
        pltpu.VMEM_SHARED((_NP,), jnp.float32),
    ]
    rpt = _NP // 16

    def body(w_hbm, dst_hbm, out_hbm, idx_d, wbuf, zb, acc):
        c = lax.axis_index("c")
        s = lax.axis_index("s")
        wid = s * 2 + c
        zero = jnp.zeros((16,), jnp.float32)

        def zloop(i, carry):
            zb[pl.ds(i * 16, 16)] = zero
            return carry

        lax.fori_loop(0, rpt // 16, zloop, 0)
        pltpu.sync_copy(zb, acc.at[pl.ds(s * rpt, rpt)])
        plsc.subcore_barrier()

        def step(i, carry):
            off = (wid * steps + i) * _CH
            pltpu.sync_copy(dst_hbm.at[pl.ds(off, _CH)], idx_d)
            pltpu.sync_copy(w_hbm.at[pl.ds(off, _CH)], wbuf)
            pltpu.sync_copy(wbuf, acc.at[idx_d], add=True)
            return carry

        lax.fori_loop(0, steps, step, 0)
        plsc.subcore_barrier()
        pltpu.sync_copy(acc.at[pl.ds(s * rpt, rpt)],
                        out_hbm.at[c, pl.ds(s * rpt, rpt)])

    return pl.kernel(
        body,
        out_type=jax.ShapeDtypeStruct((2, _NP), jnp.float32),
        mesh=_mesh(), scratch_types=scratch)


def _make_raw_pass(ep, steps):
    scratch = [
        pltpu.VMEM((_CH,), jnp.int32),
        pltpu.VMEM((_CH,), jnp.int32),
        pltpu.VMEM((_CH,), jnp.float32),
        pltpu.VMEM((_CH,), jnp.float32),
        pltpu.VMEM((_NP // 16,), jnp.float32),
        pltpu.VMEM_SHARED((_NP,), jnp.float32),
        pltpu.SemaphoreType.DMA,
    ]
    rpt = _NP // 16

    def body(u_hbm, v_hbm, s_hbm, dst_hbm, ex_hbm, den_hbm,
             idx_s, idx_d, ubuf, vbuf, zb, acc, sem):
        c = lax.axis_index("c")
        s = lax.axis_index("s")
        wid = s * 2 + c
        zero = jnp.zeros((16,), jnp.float32)

        def zloop(i, carry):
            zb[pl.ds(i * 16, 16)] = zero
            return carry

        lax.fori_loop(0, rpt // 16, zloop, 0)
        pltpu.sync_copy(zb, acc.at[pl.ds(s * rpt, rpt)])
        plsc.subcore_barrier()

        def step(i, carry):
            off = (wid * steps + i) * _CH
            pltpu.sync_copy(s_hbm.at[pl.ds(off, _CH)], idx_s)
            pltpu.sync_copy(dst_hbm.at[pl.ds(off, _CH)], idx_d)
            pltpu.async_copy(u_hbm.at[idx_d], ubuf, sem).wait()
            pltpu.async_copy(v_hbm.at[idx_s], vbuf, sem).wait()

            def comp(j, carry2):
                sl = pl.ds(j * 16, 16)
                t = ubuf[sl] + vbuf[sl]
                t = jnp.where(t >= 0.0, t, 0.2 * t)
                ubuf[sl] = jnp.exp(t)
                return carry2

            lax.fori_loop(0, _CH // 16, comp, 0)
            pltpu.sync_copy(ubuf, ex_hbm.at[pl.ds(off, _CH)])
            pltpu.sync_copy(ubuf, acc.at[idx_d], add=True)
            return carry

        lax.fori_loop(0, steps, step, 0)
        plsc.subcore_barrier()
        pltpu.sync_copy(acc.at[pl.ds(s * rpt, rpt)],
                        den_hbm.at[c, pl.ds(s * rpt, rpt)])

    return pl.kernel(
        body,
        out_type=(jax.ShapeDtypeStruct((ep,), jnp.float32),
                  jax.ShapeDtypeStruct((2, _NP), jnp.float32)),
        mesh=_mesh(), scratch_types=scratch)


_rowp = _make_row_pass(_EP1, 11, False)
_rowpw = _make_row_pass(_EP1, 11, True)
_scalar_pass = _make_scalar_pass(_EP1, 11)
_raw_pass = _make_raw_pass(_EP1, 11)


def _in_tf_kernel(x_ref, w_ref, dinv_ref, o_ref):
    o_ref[...] = jnp.dot(x_ref[...], w_ref[...],
                         preferred_element_type=jnp.float32) * dinv_ref[...]


def _mk_in_tf(din):
    return pl.pallas_call(
        _in_tf_kernel,
        in_specs=[pl.BlockSpec((_NP, din), lambda: (0, 0)),
                  pl.BlockSpec((din, _FIN), lambda: (0, 0)),
                  pl.BlockSpec((_NP, 1), lambda: (0, 0))],
        out_specs=pl.BlockSpec((_NP, _FIN), lambda: (0, 0)),
        out_shape=jax.ShapeDtypeStruct((_NP, _FIN), jnp.float32),
    )


_tf_in = _mk_in_tf(_FIN)
_tf_h = _mk_in_tf(_H)


def _combine_tf_kernel(p_ref, dinv_ref, b_ref, w_ref, x_ref, hs_ref):
    x = jnp.tanh((p_ref[0] + p_ref[1]) * dinv_ref[...] + b_ref[...])
    x_ref[...] = x
    hs_ref[...] = jnp.dot(x, w_ref[...],
                          preferred_element_type=jnp.float32) * dinv_ref[...]


_combine_h = pl.pallas_call(
    _combine_tf_kernel,
    in_specs=[pl.BlockSpec((2, _NP, _FIN), lambda: (0, 0, 0)),
              pl.BlockSpec((_NP, 1), lambda: (0, 0)),
              pl.BlockSpec((1, _FIN), lambda: (0, 0)),
              pl.BlockSpec((_FIN, _FIN), lambda: (0, 0))],
    out_specs=[pl.BlockSpec((_NP, _FIN), lambda: (0, 0)),
               pl.BlockSpec((_NP, _FIN), lambda: (0, 0))],
    out_shape=[jax.ShapeDtypeStruct((_NP, _FIN), jnp.float32),
               jax.ShapeDtypeStruct((_NP, _FIN), jnp.float32)],
)


def _final_combine_kernel(p_ref, dinv_ref, b_ref, o_ref):
    o_ref[...] = (p_ref[0] + p_ref[1]) * dinv_ref[...] + b_ref[...]


_final_combine = pl.pallas_call(
    _final_combine_kernel,
    grid=(1,),
    in_specs=[pl.BlockSpec((2, _N, _FIN), lambda i: (0, 0, 0)),
              pl.BlockSpec((_N, 1), lambda i: (0, 0)),
              pl.BlockSpec((1, _FIN), lambda i: (0, 0))],
    out_specs=pl.BlockSpec((_N, _FIN), lambda i: (0, 0)),
    out_shape=jax.ShapeDtypeStruct((_N, _FIN), jnp.float32),
)


def _mm_r_kernel(s_ref, ap_ref, o_ref):
    o_ref[...] = jnp.dot(s_ref[...], ap_ref[...],
                         preferred_element_type=jnp.float32)


def _mm_big_kernel(r_ref, st_ref, o_ref):
    o_ref[...] = jnp.dot(r_ref[...], st_ref[...],
                         preferred_element_type=jnp.float32)


def _reconstruct_adj(s_pad, st_pad, adj_pool_pad):
    r = pl.pallas_call(
        _mm_r_kernel,
        grid=(_N // _BM,),
        in_specs=[
            pl.BlockSpec((_BM, _KP), lambda i: (i, 0)),
            pl.BlockSpec((_KP, _KP), lambda i: (0, 0)),
        ],
        out_specs=pl.BlockSpec((_BM, _KP), lambda i: (i, 0)),
        out_shape=jax.ShapeDtypeStruct((_N, _KP), jnp.float32),
    )(s_pad, adj_pool_pad)
    r16 = r.astype(jnp.bfloat16)
    adj = pl.pallas_call(
        _mm_big_kernel,
        grid=(_N // _BR,),
        in_specs=[
            pl.BlockSpec((_BR, _KP), lambda i: (i, 0)),
            pl.BlockSpec((_KP, _N), lambda i: (0, 0)),
        ],
        out_specs=pl.BlockSpec((_BR, _N), lambda i: (i, 0)),
        out_shape=jax.ShapeDtypeStruct((_N, _N), jnp.float32),
    )(r16, st_pad)
    return adj


def _pad_rows(x, rows=_NP):
    return jnp.pad(x, ((0, rows - x.shape[0]),) + ((0, 0),) * (x.ndim - 1))


def kernel(nodes, edges, batch, W1, b1, W2, b2, W3, b3, W4, b4, W5, b5,
           att, w_fit, b_fit):
    src = edges[0].astype(jnp.int32)
    dst = edges[1].astype(jnp.int32)
    loop = jnp.arange(_N, dtype=jnp.int32)
    s2 = jnp.concatenate([src, loop])
    d2 = jnp.concatenate([dst, loop])
    npad = _EP1 - (_E + _N)
    s2p = jnp.concatenate([s2, jnp.full((npad,), _TRASH, jnp.int32)])
    d2p = jnp.concatenate([d2, jnp.full((npad,), _TRASH, jnp.int32)])

    ones_e = jnp.ones((_EP1,), jnp.float32)
    degp = _scalar_pass(ones_e, d2p)
    deg = degp[0, :_N] + degp[1, :_N]
    dinv = jnp.where(deg > 0, 1.0 / jnp.sqrt(deg), 0.0)
    dinv2 = _pad_rows(dinv[:, None])

    nodes_pad = _pad_rows(nodes)
    padw = _FIN - _H
    W1p = jnp.pad(W1, ((0, 0), (0, padw)))
    W2sq = jnp.pad(W2, ((0, padw), (0, padw)))
    W4sq = jnp.pad(W4, ((0, padw), (0, padw)))
    W5sq = jnp.pad(W5, ((0, padw), (0, 0)))
    b1p = jnp.pad(b1, (0, padw))[None, :]
    b2p = jnp.pad(b2, (0, padw))[None, :]
    b3p = jnp.pad(b3, (0, padw))[None, :]
    b4p = jnp.pad(b4, (0, padw))[None, :]

    hs1 = _tf_in(nodes_pad, W1p, dinv2)
    p = _row_partials(_rowp(hs1, s2p, d2p))
    x1, hs2 = _combine_h(p, dinv2, b1p, W2sq)
    p = _row_partials(_rowp(hs2, s2p, d2p))
    x2, _unused = _combine_h(p, dinv2, b2p, W2sq)
    v = x2 @ jnp.pad(att[_H:], (0, padw))

    x_q = jax.ops.segment_max(x2[:_N][s2], d2, num_segments=_N)
    u = _pad_rows((x_q @ jnp.pad(att[:_H], (0, padw)))[:, None])[:, 0]
    ex, denp = _raw_pass(u, v, s2p, d2p)
    den = denp[0] + denp[1]
    num = _row_partials(_rowpw(x2, s2p, d2p, ex))
    x_c = (num[0, :_N, :_H] + num[1, :_N, :_H]) / (den[:_N, None] + 1e-16)

    fitness = jax.nn.sigmoid(x_c @ w_fit + b_fit)
    vals, perm = jax.lax.top_k(fitness, _K)
    x_pool = x_c[perm] * vals[:, None]
    inv = jnp.full((_N,), _K, jnp.int32).at[perm].set(
        jnp.arange(_K, dtype=jnp.int32))
    col = inv[d2]
    score = ex[:_E + _N] / (den[:_N][d2] + 1e-16)
    s_sel = jnp.zeros((_N, _K + 1), jnp.float32).at[s2, col].add(score)[:, :_K]
    a_s = jax.ops.segment_sum(s_sel[dst], src, num_segments=_N)
    adj_pool = s_sel.T @ a_s

    x_out = s_sel @ x_pool

    s_pad = jnp.pad(s_sel, ((0, 0), (0, _KP - _K)))
    st_pad = jnp.pad(s_sel.T, ((0, _KP - _K), (0, 0))).astype(jnp.bfloat16)
    ap_pad = jnp.pad(adj_pool, ((0, _KP - _K), (0, _KP - _K)))
    adj_out = _reconstruct_adj(s_pad, st_pad, ap_pad)

    hs3 = _tf_h(_pad_rows(x_out), jnp.pad(W3, ((0, 0), (0, padw))), dinv2)
    p = _row_partials(_rowp(hs3, s2p, d2p))
    x3, hs4 = _combine_h(p, dinv2, b3p, W4sq)
    p = _row_partials(_rowp(hs4, s2p, d2p))
    x4, hs5 = _combine_h(p, dinv2, b4p, W5sq)
    p = _row_partials(_rowp(hs5, s2p, d2p))
    x = _final_combine(p, dinv2, b5[None, :])
    return x, adj_out[None]
